# baseline (speedup 1.0000x reference)
import math

import jax
import jax.numpy as jnp
from jax.experimental import pallas as pl

D_MODEL = 1024
D_REASON = 128
N_OPS = 32
TOP_K = 2
N_SLOTS = 16
DEPTHS = (1, 3, 8)
NEG_INF = -1e30

_OPS_END = N_OPS * D_REASON
_WV_END = _OPS_END + D_REASON
_LG_END = _WV_END + N_OPS
_SC_END = _LG_END + N_SLOTS
_WS_END = _SC_END + N_SLOTS
_CAT_ROWS = _WS_END + 1


def _dot_nt(a, b):
    return jax.lax.dot_general(
        a, b, (((1,), (1,)), ((), ())), preferred_element_type=jnp.float32)


def _dot_nn(a, b):
    return jax.lax.dot_general(
        a, b, (((1,), (0,)), ((), ())), preferred_element_type=jnp.float32)


def _prep_kernel(ops_ref, wr_ref, wd_ref, wrd_ref, wwk_ref, wv_ref,
                 slots_ref, ops_out, s_out, wd_out, small_out):
    ops = ops_ref[...]
    s = jnp.maximum(jnp.mean(jnp.abs(ops), axis=1, keepdims=True), 1e-5)
    ops_out[...] = jnp.clip(jnp.round(ops / s), -1.0, 1.0) * s
    s_out[...] = jnp.broadcast_to(s, (N_OPS, D_REASON))

    wr = wr_ref[...]
    sr = jnp.maximum(jnp.mean(jnp.abs(wr)), 1e-5)
    wr_eff = jnp.clip(jnp.round(wr / sr), -1.0, 1.0) * sr

    wd = wd_ref[...]
    sd = jnp.maximum(jnp.mean(jnp.abs(wd)), 1e-5)
    wd_out[...] = jnp.clip(jnp.round(wd / sd), -1.0, 1.0) * sd

    slots = slots_ref[...]
    small_out[0:D_REASON, :] = wv_ref[...]
    small_out[D_REASON:D_REASON + N_OPS, :] = wr_eff
    small_out[D_REASON + N_OPS:D_REASON + N_OPS + N_SLOTS, :] = \
        _dot_nn(slots, wrd_ref[...])
    small_out[D_REASON + N_OPS + N_SLOTS:, :] = _dot_nn(slots, wwk_ref[...])


def _main_kernel(x_ref, wdown_ref, wup_ref, wd_eff_ref, ops_flat_ref,
                 wsmall_ref, wwg_ref, slots_ref, out_ref):
    x = x_ref[...]
    blk = x.shape[0]
    inv_sqrt_d = 1.0 / math.sqrt(D_REASON)

    reason = _dot_nt(x, wdown_ref[...])

    wd_eff = wd_eff_ref[...]
    dl0 = jnp.sum(reason * wd_eff[0:1, :], axis=1, keepdims=True)
    dl1 = jnp.sum(reason * wd_eff[1:2, :], axis=1, keepdims=True)
    dl2 = jnp.sum(reason * wd_eff[2:3, :], axis=1, keepdims=True)
    dm = jnp.maximum(dl0, jnp.maximum(dl1, dl2))
    e0 = jnp.exp(dl0 - dm)
    e1 = jnp.exp(dl1 - dm)
    e2 = jnp.exp(dl2 - dm)
    dz = e0 + e1 + e2
    p0, p1, p2 = e0 / dz, e1 / dz, e2 / dz

    ops_flat = ops_flat_ref[...]
    wsmall = wsmall_ref[...]
    wwgate = wwg_ref[...]
    slots = slots_ref[...]

    iota_ops = jax.lax.broadcasted_iota(jnp.int32, (blk, N_OPS), 1)
    nidx_bf = (jax.lax.broadcasted_iota(
        jnp.int32, (blk, N_OPS * D_REASON), 1) // D_REASON
               ).astype(jnp.bfloat16)

    state = reason
    results = []

    for step in range(DEPTHS[-1]):
        state_bf = state.astype(jnp.bfloat16)
        ts = _dot_nt(state, wsmall)
        wv = ts[:, 0:D_REASON]
        logits = ts[:, D_REASON:D_REASON + N_OPS]
        scores = ts[:, D_REASON + N_OPS:D_REASON + N_OPS + N_SLOTS] \
            * inv_sqrt_d
        wscores = ts[:, D_REASON + N_OPS + N_SLOTS:] * inv_sqrt_d
        wgl = jnp.sum(state * wwgate, axis=1, keepdims=True)

        m0 = jnp.max(logits, axis=1, keepdims=True)
        i0 = jnp.min(jnp.where(logits == m0, iota_ops, N_OPS),
                     axis=1, keepdims=True)
        masked = jnp.where(iota_ops == i0, NEG_INF, logits)
        m1 = jnp.max(masked, axis=1, keepdims=True)
        i1 = jnp.min(jnp.where(masked == m1, iota_ops, N_OPS),
                     axis=1, keepdims=True)
        e = jnp.exp(m1 - m0)
        w0 = 1.0 / (1.0 + e)
        w1 = 1.0 - w0

        i0b = i0.astype(jnp.bfloat16)
        i1b = i1.astype(jnp.bfloat16)
        w0b = w0.astype(jnp.bfloat16)
        w1b = w1.astype(jnp.bfloat16)
        zero = jnp.bfloat16(0.0)
        cf = (jnp.where(nidx_bf == i0b, w0b, zero)
              + jnp.where(nidx_bf == i1b, w1b, zero)
              + jnp.bfloat16(1e-5 / N_OPS))
        z = cf * jnp.tile(state_bf, (1, N_OPS))
        op_out = _dot_nn(z, ops_flat)

        sm = jnp.max(scores, axis=1, keepdims=True)
        se = jnp.exp(scores - sm)
        attn = se / jnp.sum(se, axis=1, keepdims=True)
        mem = _dot_nn(attn, slots)

        wsm = jnp.max(wscores, axis=1, keepdims=True)
        wse = jnp.exp(wscores - wsm)
        aw = wse / jnp.sum(wse, axis=1, keepdims=True)
        wsig = jax.nn.sigmoid(wgl) * wv + 0.1 * _dot_nn(aw, slots)

        state = state + op_out + mem + 0.1 * wsig
        if (step + 1) in DEPTHS:
            results.append(state)

    blended = p0 * results[0] + p1 * results[1] + p2 * results[2]
    out_ref[...] = x + _dot_nt(blended, wup_ref[...])


def kernel(x, W_down, W_up, W_depth, W_router, ops, W_read, W_wkey, W_wgate,
           W_wval, slots):
    B, S, _ = x.shape
    T = B * S

    ops_tern, s_mat, wd_eff, w_small = pl.pallas_call(
        _prep_kernel,
        out_shape=(
            jax.ShapeDtypeStruct((N_OPS, D_REASON * D_REASON), jnp.float32),
            jax.ShapeDtypeStruct((N_OPS, D_REASON), jnp.float32),
            jax.ShapeDtypeStruct((len(DEPTHS), D_REASON), jnp.float32),
            jax.ShapeDtypeStruct((D_REASON + N_OPS + 2 * N_SLOTS, D_REASON),
                                 jnp.float32),
        ),
    )(ops.reshape(N_OPS, D_REASON * D_REASON), W_router, W_depth,
      W_read, W_wkey, W_wval, slots)

    ops_flat = ops_tern.reshape(N_OPS, D_REASON, D_REASON).transpose(
        0, 2, 1).reshape(N_OPS * D_REASON, D_REASON).astype(jnp.bfloat16)
    s_row = s_mat[:, 0].reshape(1, N_OPS)

    x2 = x.reshape(T, D_MODEL)
    BLK = 512
    grid = (T // BLK,)

    full = lambda shape: pl.BlockSpec(shape, lambda i: (0, 0))
    out = pl.pallas_call(
        _main_kernel,
        grid=grid,
        in_specs=[
            pl.BlockSpec((BLK, D_MODEL), lambda i: (i, 0)),
            full((D_REASON, D_MODEL)),
            full((D_MODEL, D_REASON)),
            full((len(DEPTHS), D_REASON)),
            full((_OPS_END, D_REASON)),
            full((D_REASON + N_OPS + 2 * N_SLOTS, D_REASON)),
            full((1, D_REASON)),
            full((N_SLOTS, D_REASON)),
        ],
        out_specs=pl.BlockSpec((BLK, D_MODEL), lambda i: (i, 0)),
        out_shape=jax.ShapeDtypeStruct((T, D_MODEL), jnp.float32),
    )(x2, W_down, W_up, wd_eff, ops_flat, w_small, W_wgate, slots)

    return out.reshape(B, S, D_MODEL)


# speedup vs baseline: 1.5908x; 1.5908x over previous
import math

import jax
import jax.numpy as jnp
from jax.experimental import pallas as pl

D_MODEL = 1024
D_REASON = 128
N_OPS = 32
TOP_K = 2
N_SLOTS = 16
DEPTHS = (1, 3, 8)
NEG_INF = -1e30

_OPS_END = N_OPS * D_REASON
_WV_END = _OPS_END + D_REASON
_LG_END = _WV_END + N_OPS
_SC_END = _LG_END + N_SLOTS
_WS_END = _SC_END + N_SLOTS
_CAT_ROWS = _WS_END + 1


def _dot_nt(a, b):
    return jax.lax.dot_general(
        a, b, (((1,), (1,)), ((), ())), preferred_element_type=jnp.float32)


def _dot_nn(a, b):
    return jax.lax.dot_general(
        a, b, (((1,), (0,)), ((), ())), preferred_element_type=jnp.float32)


def _prep_kernel(ops_ref, wr_ref, wd_ref, wrd_ref, wwk_ref,
                 slots_ref, ops_out, wsum_out, wd_out, wr_out, mrd_out,
                 mwk_out):
    ops = ops_ref[...]
    s = jnp.maximum(jnp.mean(jnp.abs(ops), axis=1, keepdims=True), 1e-5)
    ops_eff = jnp.clip(jnp.round(ops / s), -1.0, 1.0) * s
    ops_out[...] = ops_eff
    wsum_out[...] = (1e-5 / N_OPS) * jnp.sum(ops_eff, axis=0, keepdims=True)

    wr = wr_ref[...]
    sr = jnp.maximum(jnp.mean(jnp.abs(wr)), 1e-5)
    wr_eff = jnp.clip(jnp.round(wr / sr), -1.0, 1.0) * sr

    wd = wd_ref[...]
    sd = jnp.maximum(jnp.mean(jnp.abs(wd)), 1e-5)
    wd_out[...] = jnp.clip(jnp.round(wd / sd), -1.0, 1.0) * sd

    slots = slots_ref[...]
    wr_out[...] = wr_eff
    mrd_out[...] = _dot_nn(slots, wrd_ref[...])
    mwk_out[...] = _dot_nn(slots, wwk_ref[...])


def _main_kernel(x_ref, wdown_ref, wup_ref, wd_eff_ref, ops_flat_ref,
                 wr_eff_ref, mrd_ref, mwk_ref, wv_w_ref, wwg_ref, wsum_ref,
                 slots_ref, out_ref):
    x = x_ref[...]
    blk = x.shape[0]
    inv_sqrt_d = 1.0 / math.sqrt(D_REASON)

    reason = _dot_nt(x, wdown_ref[...])

    wd_eff = wd_eff_ref[...]
    dl0 = jnp.sum(reason * wd_eff[0:1, :], axis=1, keepdims=True)
    dl1 = jnp.sum(reason * wd_eff[1:2, :], axis=1, keepdims=True)
    dl2 = jnp.sum(reason * wd_eff[2:3, :], axis=1, keepdims=True)
    dm = jnp.maximum(dl0, jnp.maximum(dl1, dl2))
    e0 = jnp.exp(dl0 - dm)
    e1 = jnp.exp(dl1 - dm)
    e2 = jnp.exp(dl2 - dm)
    dz = e0 + e1 + e2
    p0, p1, p2 = e0 / dz, e1 / dz, e2 / dz

    ops_flat = ops_flat_ref[...]
    wr_eff = wr_eff_ref[...]
    m_read = mrd_ref[...]
    m_wkey = mwk_ref[...]
    wv_w = wv_w_ref[...]
    wwgate = wwg_ref[...]
    wsum = wsum_ref[...]
    slots = slots_ref[...]

    iota_ops = jax.lax.broadcasted_iota(
        jnp.int32, (blk, N_OPS), 1).astype(jnp.float32)
    nidx_bf = (jax.lax.broadcasted_iota(
        jnp.int32, (blk, N_OPS * D_REASON), 1) // D_REASON
               ).astype(jnp.bfloat16)

    state = reason
    results = []

    for step in range(DEPTHS[-1]):
        state_bf = state.astype(jnp.bfloat16)
        wv = _dot_nt(state, wv_w)
        logits = _dot_nt(state, wr_eff)
        scores = _dot_nt(state, m_read) * inv_sqrt_d
        wscores = _dot_nt(state, m_wkey) * inv_sqrt_d
        wgl = jnp.sum(state * wwgate, axis=1, keepdims=True)

        m0 = jnp.max(logits, axis=1, keepdims=True)
        i0 = jnp.min(jnp.where(logits == m0, iota_ops, float(N_OPS)),
                     axis=1, keepdims=True)
        masked = jnp.where(iota_ops == i0, NEG_INF, logits)
        m1 = jnp.max(masked, axis=1, keepdims=True)
        i1 = jnp.min(jnp.where(masked == m1, iota_ops, float(N_OPS)),
                     axis=1, keepdims=True)
        e = jnp.exp(m1 - m0)
        w0 = 1.0 / (1.0 + e)
        w1 = 1.0 - w0

        i0b = i0.astype(jnp.bfloat16)
        i1b = i1.astype(jnp.bfloat16)
        w0b = w0.astype(jnp.bfloat16)
        w1b = w1.astype(jnp.bfloat16)
        zero = jnp.bfloat16(0.0)
        cf = jnp.where(nidx_bf == i0b, w0b,
                       jnp.where(nidx_bf == i1b, w1b, zero))
        z = cf * jnp.tile(state_bf, (1, N_OPS))
        op_out = _dot_nn(z, ops_flat) + _dot_nt(state, wsum)

        sm = jnp.max(scores, axis=1, keepdims=True)
        se = jnp.exp(scores - sm)
        attn = se / jnp.sum(se, axis=1, keepdims=True)
        mem = _dot_nn(attn, slots)

        wsm = jnp.max(wscores, axis=1, keepdims=True)
        wse = jnp.exp(wscores - wsm)
        aw = wse / jnp.sum(wse, axis=1, keepdims=True)
        wsig = jax.nn.sigmoid(wgl) * wv + 0.1 * _dot_nn(aw, slots)

        state = state + op_out + mem + 0.1 * wsig
        if (step + 1) in DEPTHS:
            results.append(state)

    blended = p0 * results[0] + p1 * results[1] + p2 * results[2]
    out_ref[...] = x + _dot_nt(blended, wup_ref[...])


def kernel(x, W_down, W_up, W_depth, W_router, ops, W_read, W_wkey, W_wgate,
           W_wval, slots):
    B, S, _ = x.shape
    T = B * S

    ops_tern, wsum_flat, wd_eff, wr_eff, m_read, m_wkey = pl.pallas_call(
        _prep_kernel,
        out_shape=(
            jax.ShapeDtypeStruct((N_OPS, D_REASON * D_REASON), jnp.float32),
            jax.ShapeDtypeStruct((1, D_REASON * D_REASON), jnp.float32),
            jax.ShapeDtypeStruct((len(DEPTHS), D_REASON), jnp.float32),
            jax.ShapeDtypeStruct((N_OPS, D_REASON), jnp.float32),
            jax.ShapeDtypeStruct((N_SLOTS, D_REASON), jnp.float32),
            jax.ShapeDtypeStruct((N_SLOTS, D_REASON), jnp.float32),
        ),
    )(ops.reshape(N_OPS, D_REASON * D_REASON), W_router, W_depth,
      W_read, W_wkey, slots)

    ops_flat = ops_tern.reshape(N_OPS, D_REASON, D_REASON).transpose(
        0, 2, 1).reshape(N_OPS * D_REASON, D_REASON).astype(jnp.bfloat16)
    wsum = wsum_flat.reshape(D_REASON, D_REASON)

    x2 = x.reshape(T, D_MODEL)
    BLK = 512
    grid = (T // BLK,)

    full = lambda shape: pl.BlockSpec(shape, lambda i: (0, 0))
    out = pl.pallas_call(
        _main_kernel,
        grid=grid,
        in_specs=[
            pl.BlockSpec((BLK, D_MODEL), lambda i: (i, 0)),
            full((D_REASON, D_MODEL)),
            full((D_MODEL, D_REASON)),
            full((len(DEPTHS), D_REASON)),
            full((_OPS_END, D_REASON)),
            full((N_OPS, D_REASON)),
            full((N_SLOTS, D_REASON)),
            full((N_SLOTS, D_REASON)),
            full((D_REASON, D_REASON)),
            full((1, D_REASON)),
            full((D_REASON, D_REASON)),
            full((N_SLOTS, D_REASON)),
        ],
        out_specs=pl.BlockSpec((BLK, D_MODEL), lambda i: (i, 0)),
        out_shape=jax.ShapeDtypeStruct((T, D_MODEL), jnp.float32),
    )(x2, W_down, W_up, wd_eff, ops_flat, wr_eff, m_read, m_wkey, W_wval,
      W_wgate, wsum, slots)

    return out.reshape(B, S, D_MODEL)
